# baseline (speedup 1.0000x reference)
import functools

import jax
import jax.numpy as jnp
from jax import lax
from jax.experimental import pallas as pl
from jax.experimental.pallas import tpu as pltpu
from jax.experimental.pallas import tpu_sc as plsc


def _merge_table_body(probs_ref, t_ref, w_ref, y_ref):
    E = w_ref.shape[0]
    w = jnp.concatenate([probs_ref[0, c] * w_ref[c] for c in range(E)],
                        axis=1).astype(jnp.bfloat16)
    y_ref[...] = lax.dot_general(t_ref[...], w, (((1,), (1,)), ((), ())),
                                 preferred_element_type=jnp.float32)


def _merged_table(probs, tables_t, W_out):
    V = tables_t.shape[0]
    E, D, R = W_out.shape
    VB = next(b for b in (4000, 2000, 1000, 500, 250, 200, 100, 50, 25, 8, 1)
              if V % b == 0)
    return pl.pallas_call(
        _merge_table_body,
        grid=(V // VB,),
        in_specs=[
            pl.BlockSpec(memory_space=pltpu.SMEM),
            pl.BlockSpec((VB, E * R), lambda i: (i, 0)),
            pl.BlockSpec((E, D, R), lambda i: (0, 0, 0)),
        ],
        out_specs=pl.BlockSpec((VB, D), lambda i: (i, 0)),
        out_shape=jax.ShapeDtypeStruct((V, D), jnp.float32),
        compiler_params=pltpu.CompilerParams(
            allow_input_fusion=[False, True, False]),
    )(probs.reshape(1, E), tables_t, W_out)


def _make_sc_gather(B, L, D, NW):
    RPW = B // NW
    RPC = 2
    CH = RPC * L
    NCH = RPW // RPC
    mesh = plsc.VectorSubcoreMesh(core_axis_name="c", subcore_axis_name="s")

    @functools.partial(
        pl.kernel,
        out_type=jax.ShapeDtypeStruct((L, B, D), jnp.float32),
        mesh=mesh,
        scratch_types=[
            pltpu.VMEM((NCH, CH), jnp.int32),
            pltpu.VMEM((2, CH, D), jnp.float32),
            pltpu.SemaphoreType.DMA,
        ],
    )
    def gather_kernel(y_hbm, idx_hbm, out_hbm, idx_v, rows_v, sem):
        cid = lax.axis_index("c")
        sid = lax.axis_index("s")
        wid = sid * 2 + cid
        base = wid * RPW
        pltpu.sync_copy(idx_hbm.at[wid], idx_v)
        pltpu.async_copy(y_hbm.at[idx_v.at[0]], rows_v.at[0], sem)

        def body(g, carry):
            for b in range(2):
                j = g * 2 + b

                @pl.when(j + 1 < NCH)
                def _():
                    pltpu.async_copy(y_hbm.at[idx_v.at[j + 1]],
                                     rows_v.at[1 - b], sem)

                pltpu.make_async_copy(y_hbm.at[idx_v.at[j]],
                                      rows_v.at[b], sem).wait()
                r0 = base + j * RPC
                for q in range(RPC):
                    pltpu.sync_copy(rows_v.at[b, pl.ds(q * L, L)],
                                    out_hbm.at[:, r0 + q])
            return carry

        lax.fori_loop(0, NCH // 2, body, 0)

    return gather_kernel


def kernel(x, probs, tables_in, W_out):
    B, L = x.shape
    E, V, R = tables_in.shape
    D = W_out.shape[1]
    NW = 32

    tables_t = lax.reshape(tables_in.astype(jnp.bfloat16), (V, E * R),
                           dimensions=(1, 0, 2))
    y = _merged_table(probs, tables_t, W_out)
    idx = x.reshape(NW, B // (NW * 2), 2 * L).astype(jnp.int32)
    out_lbd = _make_sc_gather(B, L, D, NW)(y, idx)
    return jnp.transpose(out_lbd, (1, 0, 2))


# speedup vs baseline: 1.0793x; 1.0793x over previous
import functools

import jax
import jax.numpy as jnp
from jax import lax
from jax.experimental import pallas as pl
from jax.experimental.pallas import tpu as pltpu
from jax.experimental.pallas import tpu_sc as plsc


def _merge_table_body(probs_ref, t_ref, w_ref, y_ref):
    E = w_ref.shape[0]
    w = jnp.concatenate([probs_ref[0, c] * w_ref[c] for c in range(E)],
                        axis=1)
    y_ref[...] = lax.dot_general(t_ref[...], w, (((1,), (1,)), ((), ())),
                                 preferred_element_type=jnp.float32)


def _merged_table(probs, tables_t, W_out):
    V = tables_t.shape[0]
    E, D, R = W_out.shape
    VB = next(b for b in (4000, 2000, 1000, 500, 250, 200, 100, 50, 25, 8, 1)
              if V % b == 0)
    return pl.pallas_call(
        _merge_table_body,
        grid=(V // VB,),
        in_specs=[
            pl.BlockSpec(memory_space=pltpu.SMEM),
            pl.BlockSpec((VB, E * R), lambda i: (i, 0)),
            pl.BlockSpec((E, D, R), lambda i: (0, 0, 0)),
        ],
        out_specs=pl.BlockSpec((VB, D), lambda i: (i, 0)),
        out_shape=jax.ShapeDtypeStruct((V, D), jnp.float32),
    )(probs.reshape(1, E), tables_t, W_out)


def _make_sc_gather(B, L, D, NW):
    RPW = B // NW
    RPC = 2
    CH = RPC * L
    NCH = RPW // RPC
    mesh = plsc.VectorSubcoreMesh(core_axis_name="c", subcore_axis_name="s")

    NB = 4

    @functools.partial(
        pl.kernel,
        out_type=jax.ShapeDtypeStruct((L, B, D), jnp.float32),
        mesh=mesh,
        scratch_types=[
            pltpu.VMEM((NCH, CH), jnp.int32),
            pltpu.VMEM((NB, CH, D), jnp.float32),
            pltpu.SemaphoreType.DMA,
            pltpu.SemaphoreType.DMA,
        ],
    )
    def gather_kernel(y_hbm, idx_hbm, out_hbm, idx_v, rows_v, gsem, ssem):
        cid = lax.axis_index("c")
        sid = lax.axis_index("s")
        wid = sid * 2 + cid
        base = wid * RPW
        pltpu.sync_copy(idx_hbm.at[wid], idx_v)
        for k in range(NB - 1):
            pltpu.async_copy(y_hbm.at[idx_v.at[k]], rows_v.at[k], gsem)

        def store_wait(bb, rr):
            for q in range(RPC):
                pltpu.make_async_copy(rows_v.at[bb, pl.ds(q * L, L)],
                                      out_hbm.at[:, rr + q], ssem).wait()

        def body(g, carry):
            for b in range(NB):
                j = g * NB + b
                pltpu.make_async_copy(y_hbm.at[idx_v.at[j]],
                                      rows_v.at[b], gsem).wait()
                r0 = base + j * RPC
                for q in range(RPC):
                    pltpu.async_copy(rows_v.at[b, pl.ds(q * L, L)],
                                     out_hbm.at[:, r0 + q], ssem)
                nxt = j + NB - 1
                pb = (b + NB - 1) % NB

                @pl.when(jnp.logical_and(nxt < NCH, j >= 1))
                def _():
                    store_wait(pb, base + (j - 1) * RPC)

                @pl.when(nxt < NCH)
                def _():
                    pltpu.async_copy(y_hbm.at[idx_v.at[nxt]],
                                     rows_v.at[pb], gsem)
            return carry

        lax.fori_loop(0, NCH // NB, body, 0)
        for k in range(NB):
            j = NCH - NB + k
            store_wait(j % NB, base + j * RPC)

    return gather_kernel


def kernel(x, probs, tables_in, W_out):
    B, L = x.shape
    E, V, R = tables_in.shape
    D = W_out.shape[1]
    NW = 32

    tables_t = lax.reshape(tables_in, (V, E * R), dimensions=(1, 0, 2))
    y = _merged_table(probs, tables_t, W_out)
    idx = x.reshape(NW, B // (NW * 2), 2 * L).astype(jnp.int32)
    out_lbd = _make_sc_gather(B, L, D, NW)(y, idx)
    return jnp.transpose(out_lbd, (1, 0, 2))


# speedup vs baseline: 1.1117x; 1.0301x over previous
import functools

import jax
import jax.numpy as jnp
from jax import lax
from jax.experimental import pallas as pl
from jax.experimental.pallas import tpu as pltpu
from jax.experimental.pallas import tpu_sc as plsc


def _merge_table_body(probs_ref, t_ref, w_ref, y_ref):
    E = w_ref.shape[0]
    w = jnp.concatenate([probs_ref[0, c] * w_ref[c] for c in range(E)],
                        axis=1)
    y_ref[...] = lax.dot_general(t_ref[...], w, (((1,), (1,)), ((), ())),
                                 preferred_element_type=jnp.float32)


def _merged_table(probs, tables_t, W_out):
    V = tables_t.shape[0]
    E, D, R = W_out.shape
    VB = next(b for b in (10000, 4000, 2000, 1000, 500, 250, 200, 100, 50, 25,
                          8, 1) if V % b == 0)
    return pl.pallas_call(
        _merge_table_body,
        grid=(V // VB,),
        in_specs=[
            pl.BlockSpec(memory_space=pltpu.SMEM),
            pl.BlockSpec((VB, E * R), lambda i: (i, 0)),
            pl.BlockSpec((E, D, R), lambda i: (0, 0, 0)),
        ],
        out_specs=pl.BlockSpec((VB, D), lambda i: (i, 0)),
        out_shape=jax.ShapeDtypeStruct((V, D), jnp.float32),
    )(probs.reshape(1, E), tables_t, W_out)


def _make_sc_gather(B, L, D, NW):
    RPW = B // NW
    RPC = 2
    CH = RPC * L
    NCH = RPW // RPC
    mesh = plsc.VectorSubcoreMesh(core_axis_name="c", subcore_axis_name="s")

    NB = 8

    @functools.partial(
        pl.kernel,
        out_type=jax.ShapeDtypeStruct((L, B, D), jnp.float32),
        mesh=mesh,
        scratch_types=[
            pltpu.VMEM((NCH, CH), jnp.int32),
            pltpu.VMEM((NB, CH, D), jnp.float32),
            pltpu.SemaphoreType.DMA,
            pltpu.SemaphoreType.DMA,
        ],
    )
    def gather_kernel(y_hbm, idx_hbm, out_hbm, idx_v, rows_v, gsem, ssem):
        cid = lax.axis_index("c")
        sid = lax.axis_index("s")
        wid = sid * 2 + cid
        base = wid * RPW
        pltpu.sync_copy(idx_hbm.at[wid], idx_v)
        for k in range(NB - 1):
            pltpu.async_copy(y_hbm.at[idx_v.at[k]], rows_v.at[k], gsem)

        def store_wait(bb, rr):
            for q in range(RPC):
                pltpu.make_async_copy(rows_v.at[bb, pl.ds(q * L, L)],
                                      out_hbm.at[:, rr + q], ssem).wait()

        def body(g, carry):
            for b in range(NB):
                j = g * NB + b
                pltpu.make_async_copy(y_hbm.at[idx_v.at[j]],
                                      rows_v.at[b], gsem).wait()
                r0 = base + j * RPC
                for q in range(RPC):
                    pltpu.async_copy(rows_v.at[b, pl.ds(q * L, L)],
                                     out_hbm.at[:, r0 + q], ssem)
                nxt = j + NB - 1
                pb = (b + NB - 1) % NB

                @pl.when(jnp.logical_and(nxt < NCH, j >= 1))
                def _():
                    store_wait(pb, base + (j - 1) * RPC)

                @pl.when(nxt < NCH)
                def _():
                    pltpu.async_copy(y_hbm.at[idx_v.at[nxt]],
                                     rows_v.at[pb], gsem)
            return carry

        lax.fori_loop(0, NCH // NB, body, 0)
        for k in range(NB):
            j = NCH - NB + k
            store_wait(j % NB, base + j * RPC)

    return gather_kernel


def kernel(x, probs, tables_in, W_out):
    B, L = x.shape
    E, V, R = tables_in.shape
    D = W_out.shape[1]
    NW = 32

    tables_t = lax.reshape(tables_in, (V, E * R), dimensions=(1, 0, 2))
    y = _merged_table(probs, tables_t, W_out)
    idx = x.reshape(NW, B // (NW * 2), 2 * L).astype(jnp.int32)
    out_lbd = _make_sc_gather(B, L, D, NW)(y, idx)
    return jnp.transpose(out_lbd, (1, 0, 2))
